# Initial kernel scaffold; baseline (speedup 1.0000x reference)
import jax
import jax.numpy as jnp
from jax.experimental import pallas as pl


def kernel(images, positions, widths):
    raise NotImplementedError("write your pallas kernel here")


# speedup vs baseline: 34.3222x; 34.3222x over previous
import functools

import jax
import jax.numpy as jnp
from jax import lax
from jax.experimental import pallas as pl
from jax.experimental.pallas import tpu as pltpu
from jax.experimental.pallas import tpu_sc as plsc

H, W = 2048, 2048
P = 64
N = 4096
PW = P + 8


def _patch_kernel(images, positions):
    info = plsc.get_sparse_core_info()
    nw = info.num_cores * info.num_subcores
    per_w = N // nw
    nbuf = 2

    mesh = plsc.VectorSubcoreMesh(core_axis_name="c", subcore_axis_name="s")

    @functools.partial(
        pl.kernel,
        mesh=mesh,
        compiler_params=pltpu.CompilerParams(use_tc_tiling_on_sc=False, needs_layout_passes=False),
        out_type=jax.ShapeDtypeStruct((N, P, P), jnp.float32),
        scratch_types=[
            pltpu.VMEM((per_w,), jnp.int32),
            pltpu.VMEM((per_w,), jnp.int32),
            pltpu.SMEM((per_w, 2), jnp.int32),
            pltpu.VMEM((nbuf, P, PW), jnp.float32),
            pltpu.VMEM((nbuf, P, P), jnp.float32),
            pltpu.SemaphoreType.DMA,
            pltpu.SemaphoreType.DMA,
        ],
    )
    def k(img_hbm, rr_hbm, cc_hbm, out_hbm, rr_v, cc_v, pos_s, buf, sbuf,
          in_sem, out_sem):
        wid = lax.axis_index("s") * info.num_cores + lax.axis_index("c")
        base = wid * per_w
        pltpu.sync_copy(rr_hbm.at[pl.ds(base, per_w)], rr_v)
        pltpu.sync_copy(cc_hbm.at[pl.ds(base, per_w)], cc_v)
        for j in range(per_w // 16):
            vr = rr_v[pl.ds(16 * j, 16)]
            vc = cc_v[pl.ds(16 * j, 16)]
            for t in range(16):
                pos_s[16 * j + t, 0] = vr[t]
                pos_s[16 * j + t, 1] = vc[t]

        lane = lax.iota(jnp.int32, 16)

        def start_in(i, slot):
            r = pos_s[i, 0]
            c = pos_s[i, 1]
            c8 = pl.multiple_of((c // 8) * 8, 8)
            pltpu.make_async_copy(
                img_hbm.at[pl.ds(r, P), pl.ds(c8, PW)], buf.at[slot], in_sem
            ).start()

        def wait_in(slot):
            pltpu.make_async_copy(
                img_hbm.at[pl.ds(0, P), pl.ds(0, PW)], buf.at[slot], in_sem
            ).wait()

        def start_out(i, slot):
            pltpu.make_async_copy(
                sbuf.at[slot], out_hbm.at[base + i], out_sem
            ).start()

        def wait_out(i, slot):
            pltpu.make_async_copy(
                sbuf.at[slot], out_hbm.at[base + i], out_sem
            ).wait()

        def shuffle(i, slot):
            s = pos_s[i, 1] % 8
            src = buf.at[slot]
            dst = sbuf.at[slot]

            def row_body(row, _):
                ridx = jnp.full((16,), row, dtype=jnp.int32)
                for kk in range(P // 16):
                    cidx = s + kk * 16 + lane
                    v = plsc.load_gather(src, [ridx, cidx])
                    plsc.store_scatter(dst, [ridx, kk * 16 + lane], v)
                return 0

            lax.fori_loop(0, P, row_body, 0)

        start_in(0, 0)

        def body(i, _):
            slot = lax.rem(i, nbuf)
            nslot = lax.rem(i + 1, nbuf)

            @pl.when(i >= nbuf)
            def _():
                wait_out(i - nbuf, slot)

            @pl.when(i + 1 < per_w)
            def _():
                start_in(i + 1, nslot)

            wait_in(slot)
            shuffle(i, slot)
            start_out(i, slot)
            return 0

        lax.fori_loop(0, per_w, body, 0)
        for t in range(nbuf):
            i = per_w - nbuf + t
            wait_out(i, lax.rem(i, nbuf))

    rr = positions[:, 0]
    cc = positions[:, 1]
    return k(images, rr, cc)


def kernel(images, positions, widths):
    del widths
    return _patch_kernel(images, positions)


# speedup vs baseline: 40.2830x; 1.1737x over previous
import functools

import jax
import jax.numpy as jnp
from jax import lax
from jax.experimental import pallas as pl
from jax.experimental.pallas import tpu as pltpu
from jax.experimental.pallas import tpu_sc as plsc

H, W = 2048, 2048
P = 64
N = 4096
PW = P + 8


def _patch_kernel(images, positions):
    info = plsc.get_sparse_core_info()
    nw = info.num_cores * info.num_subcores
    per_w = N // nw
    nbuf = 4

    mesh = plsc.VectorSubcoreMesh(core_axis_name="c", subcore_axis_name="s")

    @functools.partial(
        pl.kernel,
        mesh=mesh,
        compiler_params=pltpu.CompilerParams(
            use_tc_tiling_on_sc=False, needs_layout_passes=False
        ),
        out_type=jax.ShapeDtypeStruct((N, P * P), jnp.float32),
        scratch_types=[
            pltpu.VMEM((per_w,), jnp.int32),
            pltpu.VMEM((per_w,), jnp.int32),
            pltpu.SMEM((per_w, 2), jnp.int32),
            pltpu.VMEM((nbuf, P, PW), jnp.float32),
            pltpu.VMEM((nbuf, P * P), jnp.float32),
            pltpu.SemaphoreType.DMA((nbuf,)),
            pltpu.SemaphoreType.DMA((nbuf,)),
        ],
    )
    def k(img_hbm, rr_hbm, cc_hbm, out_hbm, rr_v, cc_v, pos_s, buf, sbuf,
          in_sem, out_sem):
        wid = lax.axis_index("s") * info.num_cores + lax.axis_index("c")
        base = wid * per_w
        pltpu.sync_copy(rr_hbm.at[pl.ds(base, per_w)], rr_v)
        pltpu.sync_copy(cc_hbm.at[pl.ds(base, per_w)], cc_v)
        for j in range(per_w // 16):
            vr = rr_v[pl.ds(16 * j, 16)]
            vc = cc_v[pl.ds(16 * j, 16)]
            for t in range(16):
                pos_s[16 * j + t, 0] = vr[t]
                pos_s[16 * j + t, 1] = vc[t]

        lane = lax.iota(jnp.int32, 16)

        def start_in(i, slot):
            r = pos_s[i, 0]
            c = pos_s[i, 1]
            c8 = pl.multiple_of((c // 8) * 8, 8)
            pltpu.make_async_copy(
                img_hbm.at[pl.ds(r, P), pl.ds(c8, PW)],
                buf.at[slot],
                in_sem.at[slot],
            ).start()

        def wait_in(slot):
            pltpu.make_async_copy(
                img_hbm.at[pl.ds(0, P), pl.ds(0, PW)],
                buf.at[slot],
                in_sem.at[slot],
            ).wait()

        def start_out(i, slot):
            pltpu.make_async_copy(
                sbuf.at[slot], out_hbm.at[base + i], out_sem.at[slot]
            ).start()

        def wait_out(i, slot):
            pltpu.make_async_copy(
                sbuf.at[slot], out_hbm.at[base + i], out_sem.at[slot]
            ).wait()

        def shuffle(i, slot):
            s = pos_s[i, 1] % 8
            src = buf.at[slot]
            dst = sbuf.at[slot]
            cidx = [s + kk * 16 + lane for kk in range(P // 16)]

            def row_body(row, _):
                ridx = jnp.full((16,), row, dtype=jnp.int32)
                rbase = pl.multiple_of(row * P, P)
                for kk in range(P // 16):
                    v = plsc.load_gather(src, [ridx, cidx[kk]])
                    dst[pl.ds(rbase + kk * 16, 16)] = v
                return 0

            lax.fori_loop(0, P, row_body, 0)

        for i in range(nbuf):
            start_in(i, i)

        def body(i, _):
            slot = lax.rem(i, nbuf)
            wait_in(slot)

            @pl.when(i >= nbuf)
            def _():
                wait_out(i - nbuf, slot)

            shuffle(i, slot)
            start_out(i, slot)

            @pl.when(i + nbuf < per_w)
            def _():
                start_in(i + nbuf, slot)

            return 0

        lax.fori_loop(0, per_w, body, 0)
        for t in range(nbuf):
            i = per_w - nbuf + t
            wait_out(i, lax.rem(i, nbuf))

    rr = positions[:, 0]
    cc = positions[:, 1]
    return k(images, rr, cc)


def kernel(images, positions, widths):
    del widths
    return _patch_kernel(images, positions).reshape(N, P, P)


# speedup vs baseline: 40.7784x; 1.0123x over previous
import functools

import jax
import jax.numpy as jnp
from jax import lax
from jax.experimental import pallas as pl
from jax.experimental.pallas import tpu as pltpu
from jax.experimental.pallas import tpu_sc as plsc

H, W = 2048, 2048
P = 64
N = 4096
PW = P + 8


def _patch_kernel(images, positions):
    info = plsc.get_sparse_core_info()
    nw = info.num_cores * info.num_subcores
    per_w = N // nw
    nbuf = 4

    mesh = plsc.VectorSubcoreMesh(core_axis_name="c", subcore_axis_name="s")

    @functools.partial(
        pl.kernel,
        mesh=mesh,
        compiler_params=pltpu.CompilerParams(
            use_tc_tiling_on_sc=False, needs_layout_passes=False
        ),
        out_type=jax.ShapeDtypeStruct((N, P * P), jnp.float32),
        scratch_types=[
            pltpu.VMEM((2 * per_w,), jnp.int32),
            pltpu.SMEM((per_w, 2), jnp.int32),
            pltpu.VMEM((nbuf, P, PW), jnp.float32),
            pltpu.VMEM((nbuf, P * P), jnp.float32),
            pltpu.SemaphoreType.DMA((nbuf,)),
            pltpu.SemaphoreType.DMA((nbuf,)),
        ],
    )
    def k(img_hbm, pos_hbm, out_hbm, pos_v, pos_s, buf, sbuf,
          in_sem, out_sem):
        wid = lax.axis_index("s") * info.num_cores + lax.axis_index("c")
        base = wid * per_w
        pltpu.sync_copy(pos_hbm.at[pl.ds(2 * base, 2 * per_w)], pos_v)
        for j in range(per_w // 8):
            v = pos_v[pl.ds(16 * j, 16)]
            for t in range(8):
                pos_s[8 * j + t, 0] = v[2 * t]
                pos_s[8 * j + t, 1] = v[2 * t + 1]

        lane = lax.iota(jnp.int32, 16)

        def start_in(i, slot):
            r = pos_s[i, 0]
            c = pos_s[i, 1]
            c8 = pl.multiple_of((c // 8) * 8, 8)
            pltpu.make_async_copy(
                img_hbm.at[pl.ds(r, P), pl.ds(c8, PW)],
                buf.at[slot],
                in_sem.at[slot],
            ).start()

        def wait_in(slot):
            pltpu.make_async_copy(
                img_hbm.at[pl.ds(0, P), pl.ds(0, PW)],
                buf.at[slot],
                in_sem.at[slot],
            ).wait()

        def start_out(i, slot):
            pltpu.make_async_copy(
                sbuf.at[slot], out_hbm.at[base + i], out_sem.at[slot]
            ).start()

        def wait_out(i, slot):
            pltpu.make_async_copy(
                sbuf.at[slot], out_hbm.at[base + i], out_sem.at[slot]
            ).wait()

        def shuffle(i, slot):
            s = pos_s[i, 1] % 8
            src = buf.at[slot]
            dst = sbuf.at[slot]
            cidx = [s + kk * 16 + lane for kk in range(P // 16)]

            def row_body(q, _):
                for u in range(2):
                    row = 2 * q + u
                    ridx = jnp.full((16,), row, dtype=jnp.int32)
                    rbase = pl.multiple_of(row * P, P)
                    for kk in range(P // 16):
                        v = plsc.load_gather(src, [ridx, cidx[kk]])
                        dst[pl.ds(rbase + kk * 16, 16)] = v
                return 0

            lax.fori_loop(0, P // 2, row_body, 0)

        for i in range(nbuf):
            start_in(i, i)

        def body(i, _):
            slot = lax.rem(i, nbuf)
            wait_in(slot)

            @pl.when(i >= nbuf)
            def _():
                wait_out(i - nbuf, slot)

            shuffle(i, slot)
            start_out(i, slot)

            @pl.when(i + nbuf < per_w)
            def _():
                start_in(i + nbuf, slot)

            return 0

        lax.fori_loop(0, per_w, body, 0)
        for t in range(nbuf):
            i = per_w - nbuf + t
            wait_out(i, lax.rem(i, nbuf))

    return k(images, positions.reshape(-1))


def kernel(images, positions, widths):
    del widths
    return _patch_kernel(images, positions).reshape(N, P, P)


# speedup vs baseline: 48.0503x; 1.1783x over previous
import functools

import jax
import jax.numpy as jnp
from jax import lax
from jax.experimental import pallas as pl
from jax.experimental.pallas import tpu as pltpu
from jax.experimental.pallas import tpu_sc as plsc

H, W = 2048, 2048
P = 64
N = 4096
PW = P + 8


def _patch_kernel(images, positions):
    info = plsc.get_sparse_core_info()
    nw = info.num_cores * info.num_subcores
    per_w = N // nw
    nbuf = 6

    mesh = plsc.VectorSubcoreMesh(core_axis_name="c", subcore_axis_name="s")

    @functools.partial(
        pl.kernel,
        mesh=mesh,
        compiler_params=pltpu.CompilerParams(
            use_tc_tiling_on_sc=False, needs_layout_passes=False
        ),
        out_type=jax.ShapeDtypeStruct((N, P, P), jnp.float32),
        scratch_types=[
            pltpu.VMEM((2 * per_w,), jnp.int32),
            pltpu.SMEM((per_w, 2), jnp.int32),
            pltpu.VMEM((nbuf, P, PW), jnp.float32),
            pltpu.VMEM((nbuf, P, P), jnp.float32),
            pltpu.SemaphoreType.DMA((nbuf,)),
            pltpu.SemaphoreType.DMA((nbuf,)),
        ],
    )
    def k(img_hbm, pos_hbm, out_hbm, pos_v, pos_s, buf, sbuf,
          in_sem, out_sem):
        wid = lax.axis_index("s") * info.num_cores + lax.axis_index("c")
        base = wid * per_w
        pltpu.sync_copy(pos_hbm.at[pl.ds(2 * base, 2 * per_w)], pos_v)
        for j in range(per_w // 8):
            v = pos_v[pl.ds(16 * j, 16)]
            for t in range(8):
                pos_s[8 * j + t, 0] = v[2 * t]
                pos_s[8 * j + t, 1] = v[2 * t + 1]

        lane = lax.iota(jnp.int32, 16)

        def start_in(i, slot):
            r = pos_s[i, 0]
            c = pos_s[i, 1]
            c8 = pl.multiple_of((c // 8) * 8, 8)
            pltpu.make_async_copy(
                img_hbm.at[pl.ds(r, P), pl.ds(c8, PW)],
                buf.at[slot],
                in_sem.at[slot],
            ).start()

        def wait_in(slot):
            pltpu.make_async_copy(
                img_hbm.at[pl.ds(0, P), pl.ds(0, PW)],
                buf.at[slot],
                in_sem.at[slot],
            ).wait()

        def start_out(i, slot):
            pltpu.make_async_copy(
                sbuf.at[slot], out_hbm.at[base + i], out_sem.at[slot]
            ).start()

        def wait_out(i, slot):
            pltpu.make_async_copy(
                sbuf.at[slot], out_hbm.at[base + i], out_sem.at[slot]
            ).wait()

        def shuffle(i, slot):
            s = pos_s[i, 1] % 8
            src = buf.at[slot]
            dst = sbuf.at[slot]
            cidx = [s + kk * 16 + lane for kk in range(P // 16)]

            @functools.partial(plsc.parallel_loop, 0, P, unroll=4)
            def row_body(row):
                ridx = jnp.full((16,), row, dtype=jnp.int32)
                for kk in range(P // 16):
                    v = plsc.load_gather(src, [ridx, cidx[kk]])
                    dst[row, pl.ds(kk * 16, 16)] = v

        for i in range(nbuf):
            start_in(i, i)

        def body(i, _):
            slot = lax.rem(i, nbuf)
            wait_in(slot)

            @pl.when(i >= nbuf)
            def _():
                wait_out(i - nbuf, slot)

            shuffle(i, slot)
            start_out(i, slot)

            @pl.when(i + nbuf < per_w)
            def _():
                start_in(i + nbuf, slot)

            return 0

        lax.fori_loop(0, per_w, body, 0)
        for t in range(nbuf):
            i = per_w - nbuf + t
            wait_out(i, lax.rem(i, nbuf))

    return k(images, positions.reshape(-1))


def kernel(images, positions, widths):
    del widths
    return _patch_kernel(images, positions)
